# Initial kernel scaffold; baseline (speedup 1.0000x reference)
import functools

import jax
import jax.numpy as jnp
from jax import lax
from jax.experimental import pallas as pl
from jax.experimental.pallas import tpu as pltpu
from jax.experimental.pallas import tpu_sc as plsc

B = 128
L = 4096
D = 64
NC = 2
NS = 16
NW = NC * NS
C = L // NW


def _body(x_hbm, pos_hbm, emb_hbm, pe_hbm, out_hbm, pidx_v, pe_v, idx_v, rows_v, sem):
    wid = lax.axis_index("s") * NC + lax.axis_index("c")
    l0 = wid * C

    pltpu.sync_copy(pos_hbm.at[pl.ds(l0, C)], pidx_v)
    pltpu.async_copy(pe_hbm.at[pidx_v], pe_v, sem).wait()

    def batch_body(b, carry):
        pltpu.sync_copy(x_hbm.at[b, pl.ds(l0, C)], idx_v)
        pltpu.async_copy(emb_hbm.at[idx_v], rows_v, sem).wait()

        def row_body(i, c2):
            for j in range(D // 16):
                s = pl.ds(j * 16, 16)
                rows_v[i, s] = rows_v[i, s] + pe_v[i, s]
            return c2

        lax.fori_loop(0, C, row_body, 0, unroll=2)
        pltpu.sync_copy(rows_v, out_hbm.at[b, pl.ds(l0, C)])
        return carry

    lax.fori_loop(0, B, batch_body, 0)


@jax.jit
def _run(x, emb_table, pe_table, pos_flat):
    mesh = plsc.VectorSubcoreMesh(core_axis_name="c", subcore_axis_name="s")
    f = functools.partial(
        pl.kernel,
        out_type=jax.ShapeDtypeStruct((B, L, D), jnp.float32),
        mesh=mesh,
        scratch_types=[
            pltpu.VMEM((C,), jnp.int32),
            pltpu.VMEM((C, D), jnp.float32),
            pltpu.VMEM((C,), jnp.int32),
            pltpu.VMEM((C, D), jnp.float32),
            pltpu.SemaphoreType.DMA,
        ],
    )(_body)
    return f(x, pos_flat, emb_table, pe_table)


def kernel(x, emb_table, pe_table, positional_tokens):
    return _run(x, emb_table, pe_table, positional_tokens.reshape(L))


# speedup vs baseline: 2.2191x; 2.2191x over previous
import functools

import jax
import jax.numpy as jnp
from jax import lax
from jax.experimental import pallas as pl
from jax.experimental.pallas import tpu as pltpu
from jax.experimental.pallas import tpu_sc as plsc

B = 128
L = 4096
D = 64
NC = 2
NS = 16
NW = NC * NS
C = L // NW


def _body(x_hbm, pos_hbm, emb_hbm, pe_hbm, out_hbm, pidx_v, pe_v, idx_v, rows_v, sem):
    wid = lax.axis_index("s") * NC + lax.axis_index("c")
    l0 = wid * C

    pltpu.sync_copy(pos_hbm.at[pl.ds(l0, C)], pidx_v)
    pltpu.async_copy(pe_hbm.at[pidx_v], pe_v, sem).wait()

    def batch_body(b, carry):
        pltpu.sync_copy(x_hbm.at[b, pl.ds(l0, C)], idx_v)
        pltpu.async_copy(emb_hbm.at[idx_v], rows_v, sem).wait()

        def row_body(i, c2):
            for j in range(D // 16):
                s = pl.ds(j * 16, 16)
                rows_v[i, s] = rows_v[i, s] + pe_v[i, s]
            return c2

        lax.fori_loop(0, C, row_body, 0, unroll=2)
        pltpu.sync_copy(rows_v, out_hbm.at[b, pl.ds(l0, C)])
        return carry

    lax.fori_loop(0, B, batch_body, 0)


@jax.jit
def _run(x, emb_table, pe_table, pos_flat):
    mesh = plsc.VectorSubcoreMesh(core_axis_name="c", subcore_axis_name="s")
    f = functools.partial(
        pl.kernel,
        out_type=jax.ShapeDtypeStruct((B, L, D), jnp.float32),
        mesh=mesh,
        scratch_types=[
            pltpu.VMEM((C,), jnp.int32),
            pltpu.VMEM((C, D), jnp.float32),
            pltpu.VMEM((C,), jnp.int32),
            pltpu.VMEM((C, D), jnp.float32),
            pltpu.SemaphoreType.DMA,
        ],
        compiler_params=pltpu.CompilerParams(use_tc_tiling_on_sc=False),
    )(_body)
    return f(x, pos_flat, emb_table, pe_table)


def kernel(x, emb_table, pe_table, positional_tokens):
    return _run(x, emb_table, pe_table, positional_tokens.reshape(L))


# speedup vs baseline: 3.6892x; 1.6624x over previous
import functools

import jax
import jax.numpy as jnp
from jax import lax
from jax.experimental import pallas as pl
from jax.experimental.pallas import tpu as pltpu
from jax.experimental.pallas import tpu_sc as plsc

B = 128
L = 4096
D = 64
NC = 2
NS = 16
NW = NC * NS
C = L // NW
NBUF = 4
AHEAD = 2


def _body(x_hbm, pos_hbm, emb_hbm, pe_hbm, out_hbm,
          idx_all, pidx_v, pe_v, rows, sem_g, sem_s):
    wid = lax.axis_index("s") * NC + lax.axis_index("c")
    l0 = wid * C

    pltpu.sync_copy(x_hbm.at[:, pl.ds(l0, C)], idx_all)
    pltpu.sync_copy(pos_hbm.at[pl.ds(l0, C)], pidx_v)
    pltpu.async_copy(pe_hbm.at[pidx_v], pe_v, sem_g[0]).wait()

    def gather(b, q):
        pltpu.async_copy(emb_hbm.at[idx_all.at[b]], rows.at[q], sem_g[q])

    def wait_gather(b, q):
        pltpu.make_async_copy(emb_hbm.at[idx_all.at[b]], rows.at[q], sem_g[q]).wait()

    def store(b, q):
        pltpu.async_copy(rows.at[q], out_hbm.at[b, pl.ds(l0, C)], sem_s[q])

    def wait_store(b, q):
        pltpu.make_async_copy(rows.at[q], out_hbm.at[b, pl.ds(l0, C)], sem_s[q]).wait()

    for b in range(AHEAD):
        gather(b, b)

    def ring_body(t, carry):
        for p in range(NBUF):
            b = NBUF * t + p
            q2 = (p + AHEAD) % NBUF
            nb = b + AHEAD

            @pl.when(nb < B)
            def _issue():
                @pl.when(b >= AHEAD)
                def _drain():
                    wait_store(b - AHEAD, q2)
                gather(nb, q2)

            wait_gather(b, p)

            def row_body(i, c2):
                for j in range(D // 16):
                    s = pl.ds(j * 16, 16)
                    rows[p, i, s] = rows[p, i, s] + pe_v[i, s]
                return c2

            lax.fori_loop(0, C, row_body, 0, unroll=2)
            store(b, p)
        return carry

    lax.fori_loop(0, B // NBUF, ring_body, 0)

    for p in range(NBUF):
        wait_store(B - NBUF + p, p)


@jax.jit
def _run(x, emb_table, pe_table, pos_flat):
    mesh = plsc.VectorSubcoreMesh(core_axis_name="c", subcore_axis_name="s")
    f = functools.partial(
        pl.kernel,
        out_type=jax.ShapeDtypeStruct((B, L, D), jnp.float32),
        mesh=mesh,
        scratch_types=[
            pltpu.VMEM((B, C), jnp.int32),
            pltpu.VMEM((C,), jnp.int32),
            pltpu.VMEM((C, D), jnp.float32),
            pltpu.VMEM((NBUF, C, D), jnp.float32),
            [pltpu.SemaphoreType.DMA] * NBUF,
            [pltpu.SemaphoreType.DMA] * NBUF,
        ],
        compiler_params=pltpu.CompilerParams(use_tc_tiling_on_sc=False),
    )(_body)
    return f(x, pos_flat, emb_table, pe_table)


def kernel(x, emb_table, pe_table, positional_tokens):
    return _run(x, emb_table, pe_table, positional_tokens.reshape(L))


# speedup vs baseline: 5.8036x; 1.5731x over previous
import functools

import jax
import jax.numpy as jnp
from jax import lax
from jax.experimental import pallas as pl
from jax.experimental.pallas import tpu as pltpu
from jax.experimental.pallas import tpu_sc as plsc

B = 128
L = 4096
D = 64
DP = 128
NC = 2
NS = 16
NW = NC * NS
C = L // NW
CB = C // 8
NBUF = 4
SBUF = 2
AHEAD = 2


def _body(x_hbm, pos_hbm, emb_hbm, pe_hbm, out_hbm,
          idx_all, pe_v, rows, srows, sem_g, sem_s):
    wid = lax.axis_index("s") * NC + lax.axis_index("c")
    l0 = wid * C
    blk0 = wid * CB

    pltpu.sync_copy(pos_hbm.at[pl.ds(l0, C)], idx_all.at[0])
    pltpu.async_copy(pe_hbm.at[idx_all.at[0]], rows.at[0], sem_g[0]).wait()

    def pe_copy(i, c2):
        for j in range(D // 16):
            sl = pl.ds(j * 16, 16)
            pe_v[i, sl] = rows[0, i, sl]
        return c2

    lax.fori_loop(0, C, pe_copy, 0, unroll=2)

    pltpu.sync_copy(x_hbm.at[:, pl.ds(l0, C)], idx_all)

    def gather(b, q):
        pltpu.async_copy(emb_hbm.at[idx_all.at[b]], rows.at[q], sem_g[q])

    def wait_gather(b, q):
        pltpu.make_async_copy(emb_hbm.at[idx_all.at[b]], rows.at[q], sem_g[q]).wait()

    def store(b, s):
        pltpu.async_copy(srows.at[s], out_hbm.at[b, pl.ds(blk0, CB)], sem_s[s])

    def wait_store(b, s):
        pltpu.make_async_copy(srows.at[s], out_hbm.at[b, pl.ds(blk0, CB)],
                              sem_s[s]).wait()

    for b in range(AHEAD):
        gather(b, b % NBUF)

    def ring_body(t, carry):
        for p in range(NBUF):
            b = NBUF * t + p
            s = p % SBUF
            nb = b + AHEAD
            q2 = (p + AHEAD) % NBUF

            @pl.when(nb < B)
            def _issue():
                gather(nb, q2)

            wait_gather(b, p)

            @pl.when(b >= SBUF)
            def _drain():
                wait_store(b - SBUF, s)

            def blk_body(g, c2):
                for r in range(8):
                    for j in range(D // 16):
                        sl = pl.ds(j * 16, 16)
                        srows[s, g, r, sl] = rows[p, g * 8 + r, sl] + pe_v[g * 8 + r, sl]
                return c2

            lax.fori_loop(0, CB, blk_body, 0)
            store(b, s)
        return carry

    lax.fori_loop(0, B // NBUF, ring_body, 0)

    for s in range(SBUF):
        wait_store(B - SBUF + s, s)


@jax.jit
def _run(x, emb_pad, pe_pad, pos_flat):
    mesh = plsc.VectorSubcoreMesh(core_axis_name="c", subcore_axis_name="s")
    f = functools.partial(
        pl.kernel,
        out_type=jax.ShapeDtypeStruct((B, L // 8, 8, D), jnp.float32),
        mesh=mesh,
        scratch_types=[
            pltpu.VMEM((B, C), jnp.int32),
            pltpu.VMEM((C, D), jnp.float32),
            pltpu.VMEM((NBUF, C, DP), jnp.float32),
            pltpu.VMEM((SBUF, CB, 8, D), jnp.float32),
            [pltpu.SemaphoreType.DMA] * NBUF,
            [pltpu.SemaphoreType.DMA] * SBUF,
        ],
        compiler_params=pltpu.CompilerParams(use_tc_tiling_on_sc=True),
    )(_body)
    return f(x, pos_flat, emb_pad, pe_pad)


def kernel(x, emb_table, pe_table, positional_tokens):
    pad = ((0, 0), (0, DP - D))
    out = _run(x, jnp.pad(emb_table, pad), jnp.pad(pe_table, pad),
               positional_tokens.reshape(L))
    return out.reshape(B, L, D)
